# baseline (speedup 1.0000x reference)
import functools

import jax
import jax.numpy as jnp
from jax.experimental import pallas as pl
from jax.experimental.pallas import tpu as pltpu

N = 10000
D = 128
BM = 200


def _fused_kernel(a1_ref, a2_ref, ego_ref, wt_ref, b_ref, g_ref, beta_ref,
                  out_ref):
    i = pl.program_id(0)

    def body(a_ref, row0):
        side = jnp.dot(a_ref[:], ego_ref[:], preferred_element_type=jnp.float32)
        hi = side + ego_ref[pl.ds(row0, BM), :]
        y = jnp.dot(hi, wt_ref[:], preferred_element_type=jnp.float32) + b_ref[:]
        y = jnp.where(y >= 0, y, 0.01 * y)
        mu = jnp.mean(y, axis=-1, keepdims=True)
        var = jnp.mean((y - mu) ** 2, axis=-1, keepdims=True)
        return (y - mu) * jax.lax.rsqrt(var + 1e-5) * g_ref[:] + beta_ref[:]

    out_ref[pl.ds(0, BM), :] = body(a1_ref, i * 2 * BM)
    out_ref[pl.ds(BM, BM), :] = body(a2_ref, i * 2 * BM + BM)


@functools.partial(jax.jit, static_argnames=())
def _run(ego, A_in, wt, b, gamma, beta):
    return pl.pallas_call(
        _fused_kernel,
        grid=(N // (2 * BM),),
        in_specs=[
            pl.BlockSpec((BM, N), lambda i: (2 * i, 0)),
            pl.BlockSpec((BM, N), lambda i: (2 * i + 1, 0)),
            pl.BlockSpec((N, D), lambda i: (0, 0)),
            pl.BlockSpec((D, D), lambda i: (0, 0)),
            pl.BlockSpec((1, D), lambda i: (0, 0)),
            pl.BlockSpec((1, D), lambda i: (0, 0)),
            pl.BlockSpec((1, D), lambda i: (0, 0)),
        ],
        out_specs=pl.BlockSpec((2 * BM, D), lambda i: (i, 0)),
        out_shape=jax.ShapeDtypeStruct((N, D), jnp.float32),
        compiler_params=pltpu.CompilerParams(
            dimension_semantics=("parallel",),
        ),
    )(A_in, A_in, ego, wt, b, gamma, beta)


def kernel(ego_embeddings, A_in, all_layers, lamda, alpha, l, W_lin, b_lin,
           ln_gamma, ln_beta):
    del all_layers, lamda, alpha, l
    wt = W_lin.T
    b = b_lin.reshape(1, D)
    gamma = ln_gamma.reshape(1, D)
    beta = ln_beta.reshape(1, D)
    return _run(ego_embeddings, A_in, wt, b, gamma, beta)


# speedup vs baseline: 1.1076x; 1.1076x over previous
import functools

import jax
import jax.numpy as jnp
from jax.experimental import pallas as pl
from jax.experimental.pallas import tpu as pltpu

N = 10000
D = 128
BM = 400


def _fused_kernel(a_ref, ego_ref, wt_ref, b_ref, g_ref, beta_ref, out_ref):
    i = pl.program_id(0)
    side = jnp.dot(a_ref[:], ego_ref[:], preferred_element_type=jnp.float32)
    hi = side + ego_ref[pl.ds(i * BM, BM), :]
    y = jnp.dot(hi, wt_ref[:], preferred_element_type=jnp.float32) + b_ref[:]
    y = jnp.where(y >= 0, y, 0.01 * y)
    mu = jnp.mean(y, axis=-1, keepdims=True)
    var = jnp.mean((y - mu) ** 2, axis=-1, keepdims=True)
    out_ref[:] = (y - mu) * jax.lax.rsqrt(var + 1e-5) * g_ref[:] + beta_ref[:]


@functools.partial(jax.jit, static_argnames=())
def _run(ego, A_in, wt, b, gamma, beta):
    return pl.pallas_call(
        _fused_kernel,
        grid=(N // BM,),
        in_specs=[
            pl.BlockSpec((BM, N), lambda i: (i, 0)),
            pl.BlockSpec((N, D), lambda i: (0, 0)),
            pl.BlockSpec((D, D), lambda i: (0, 0)),
            pl.BlockSpec((1, D), lambda i: (0, 0)),
            pl.BlockSpec((1, D), lambda i: (0, 0)),
            pl.BlockSpec((1, D), lambda i: (0, 0)),
        ],
        out_specs=pl.BlockSpec((BM, D), lambda i: (i, 0)),
        out_shape=jax.ShapeDtypeStruct((N, D), jnp.float32),
        compiler_params=pltpu.CompilerParams(
            dimension_semantics=("parallel",),
        ),
    )(A_in, ego, wt, b, gamma, beta)


def kernel(ego_embeddings, A_in, all_layers, lamda, alpha, l, W_lin, b_lin,
           ln_gamma, ln_beta):
    del all_layers, lamda, alpha, l
    wt = W_lin.T
    b = b_lin.reshape(1, D)
    gamma = ln_gamma.reshape(1, D)
    beta = ln_beta.reshape(1, D)
    return _run(ego_embeddings, A_in, wt, b, gamma, beta)
